# baseline (speedup 1.0000x reference)
import functools

import jax
import jax.numpy as jnp
from jax import lax
import jax.experimental.pallas as pl
from jax.experimental.pallas import tpu as pltpu
from jax.experimental.pallas import tpu_sc as plsc

_N = 10000
_E = 320000
_D = 128
_R = 8
_C = 16

_K = 128
_NI = 20
_EPW = _K * 4 * _NI
_EHALF = 16 * _EPW
_EPAD = 2 * _EHALF
_AROWS = 10240
_TROWS = _AROWS // 16

_RB = 1000
_GRID = _N // _RB


def _sc_body(tbl_ref, idx_ref, dst_ref, out_ref,
             ib0, ib1, ib2, ib3, db0, db1, db2, db3, rows0, rows1, acc,
             is0, is1, is2, is3, gs0, gs1, ss0, ss1):
    core = lax.axis_index("c")
    s = lax.axis_index("s")
    ebase = s * _EPW
    ni = _NI
    ibufs = [ib0, ib1, ib2, ib3]
    dbufs = [db0, db1, db2, db3]
    isems = [is0, is1, is2, is3]
    rows = [rows0, rows1]
    gsems = [gs0, gs1]
    ssems = [ss0, ss1]

    @pl.when(core == 0)
    def _only_core0():
        _sc_work(tbl_ref, idx_ref, dst_ref, out_ref, ibufs, dbufs, rows,
                 acc, isems, gsems, ssems, s, ebase, ni)


def _sc_work(tbl_ref, idx_ref, dst_ref, out_ref, ibufs, dbufs, rows, acc,
             isems, gsems, ssems, s, ebase, ni):
    rows0 = rows[0]
    def _zrow(r, carry):
        for l in range(8):
            rows0[r, pl.ds(l * 16, 16)] = jnp.zeros((16,), jnp.float32)
        return carry

    lax.fori_loop(0, _K, _zrow, 0)
    for kk in range(_TROWS // _K):
        pltpu.sync_copy(rows0, acc.at[pl.ds(s * _TROWS + kk * _K, _K)])
    plsc.subcore_barrier()

    def fire_idx(slot, ch):
        off = ebase + ch * _K
        pltpu.async_copy(idx_ref.at[pl.ds(off, _K)], ibufs[slot], isems[slot])
        pltpu.async_copy(dst_ref.at[pl.ds(off, _K)], dbufs[slot], isems[slot])

    def wait_idx(slot):
        pltpu.make_async_copy(
            idx_ref.at[pl.ds(0, _K)], ibufs[slot], isems[slot]).wait()
        pltpu.make_async_copy(
            dst_ref.at[pl.ds(0, _K)], dbufs[slot], isems[slot]).wait()

    def fire_g(slot, pr):
        pltpu.async_copy(tbl_ref.at[ibufs[slot]], rows[pr], gsems[pr])

    def wait_g(pr):
        pltpu.make_async_copy(tbl_ref.at[ibufs[0]], rows[pr],
                              gsems[pr]).wait()

    def fire_s(slot, pr):
        pltpu.async_copy(rows[pr], acc.at[dbufs[slot]], ssems[pr], add=True)

    def drain_s(pr):
        pltpu.make_async_copy(rows[pr], acc.at[dbufs[0]], ssems[pr]).wait()

    fire_idx(0, 0)
    fire_idx(1, 1)
    wait_idx(0)
    fire_g(0, 0)

    def body(i, carry):
        cb = 4 * i
        for j in range(4):
            pj = j % 2
            if j < 2:
                fire_idx((j + 2) % 4, cb + j + 2)
            else:
                @pl.when(i < ni - 1)
                def _(jj=j):
                    fire_idx((jj + 2) % 4, cb + jj + 2)
            if j == 0:
                @pl.when(i > 0)
                def _():
                    drain_s(1)
                wait_idx(1)
                fire_g(1, 1)
            elif j < 3:
                drain_s(1 - pj)
                wait_idx((j + 1) % 4)
                fire_g((j + 1) % 4, 1 - pj)
            else:
                @pl.when(i < ni - 1)
                def _():
                    drain_s(0)
                    wait_idx(0)
                    fire_g(0, 0)
            wait_g(pj)
            fire_s(j, pj)
        return carry

    lax.fori_loop(0, ni, body, 0)
    drain_s(0)
    drain_s(1)
    plsc.subcore_barrier()
    pltpu.sync_copy(acc.at[pl.ds(s * _TROWS, _TROWS)],
                    out_ref.at[pl.ds(s * _TROWS, _TROWS)])


_sc_segsum = pl.kernel(
    _sc_body,
    out_type=jax.ShapeDtypeStruct((_AROWS, _D), jnp.float32),
    mesh=plsc.VectorSubcoreMesh(core_axis_name="c", subcore_axis_name="s"),
    scratch_types=(
        [pltpu.VMEM((_K,), jnp.int32)] * 8
        + [pltpu.VMEM((_K, _D), jnp.float32)] * 2
        + [pltpu.VMEM_SHARED((_AROWS, _D), jnp.float32)]
        + [pltpu.SemaphoreType.DMA] * 8
    ),
)


def _mm1_body(x_ref, w_ref, b_ref, tbl_ref, sl_ref):
    res = jnp.dot(x_ref[...], w_ref[...],
                  preferred_element_type=jnp.float32) + b_ref[...]
    for r in range(_R):
        tbl_ref[r] = res[:, r * _D:(r + 1) * _D]
    sl_ref[...] = res[:, _R * _D:]


def _mm2_body(acca_ref, accb_ref, sl_ref, w_ref, b_ref, tbl_ref, sl2_ref):
    x = jnp.maximum(acca_ref[...] + accb_ref[...] + sl_ref[...], 0.0)
    res = jnp.dot(x, w_ref[...],
                  preferred_element_type=jnp.float32) + b_ref[...]
    for r in range(_R):
        tbl_ref[r] = res[:, r * _D:(r + 1) * _D]
    sl2_ref[...] = res[:, _R * _D:]


def _comb_body(acca_ref, accb_ref, sl_ref, out_ref):
    y = jnp.maximum(acca_ref[...] + accb_ref[...] + sl_ref[...], 0.0)
    out_ref[...] = jnp.sum(y, axis=0, keepdims=True)[None]


def _head_body(p_ref, wc_ref, bc_ref, out_ref):
    hg = jnp.sum(p_ref[..., 0, :], axis=0, keepdims=True) * (1.0 / _N)
    logits = jnp.dot(hg, wc_ref[...],
                     preferred_element_type=jnp.float32) + bc_ref[...]
    out_ref[...] = jax.nn.softmax(logits, axis=1)


_WCOLS = _R * _D + _D

_mm1 = pl.pallas_call(
    _mm1_body,
    grid=(_GRID,),
    in_specs=[
        pl.BlockSpec((_RB, _D), lambda i: (i, 0)),
        pl.BlockSpec((_D, _WCOLS), lambda i: (0, 0)),
        pl.BlockSpec((1, _WCOLS), lambda i: (0, 0)),
    ],
    out_specs=[
        pl.BlockSpec((_R, _RB, _D), lambda i: (0, i, 0)),
        pl.BlockSpec((_RB, _D), lambda i: (i, 0)),
    ],
    out_shape=[
        jax.ShapeDtypeStruct((_R, _N, _D), jnp.float32),
        jax.ShapeDtypeStruct((_N, _D), jnp.float32),
    ],
)

_mm2 = pl.pallas_call(
    _mm2_body,
    grid=(_GRID,),
    in_specs=[
        pl.BlockSpec((_RB, _D), lambda i: (i, 0)),
        pl.BlockSpec((_RB, _D), lambda i: (i, 0)),
        pl.BlockSpec((_RB, _D), lambda i: (i, 0)),
        pl.BlockSpec((_D, _WCOLS), lambda i: (0, 0)),
        pl.BlockSpec((1, _WCOLS), lambda i: (0, 0)),
    ],
    out_specs=[
        pl.BlockSpec((_R, _RB, _D), lambda i: (0, i, 0)),
        pl.BlockSpec((_RB, _D), lambda i: (i, 0)),
    ],
    out_shape=[
        jax.ShapeDtypeStruct((_R, _N, _D), jnp.float32),
        jax.ShapeDtypeStruct((_N, _D), jnp.float32),
    ],
)

_comb = pl.pallas_call(
    _comb_body,
    grid=(_GRID,),
    in_specs=[
        pl.BlockSpec((_RB, _D), lambda i: (i, 0)),
        pl.BlockSpec((_RB, _D), lambda i: (i, 0)),
        pl.BlockSpec((_RB, _D), lambda i: (i, 0)),
    ],
    out_specs=pl.BlockSpec((1, 1, _D), lambda i: (i, 0, 0)),
    out_shape=jax.ShapeDtypeStruct((_GRID, 1, _D), jnp.float32),
)

_head = pl.pallas_call(
    _head_body,
    out_shape=jax.ShapeDtypeStruct((1, _C), jnp.float32),
)


def kernel(h, edge_index, rel_types, W1, loop1, b1, W2, loop2, b2, Wc, bc):
    src = edge_index[0].astype(jnp.int32)
    dst = edge_index[1].astype(jnp.int32)
    rel = rel_types.astype(jnp.int32)
    idx = rel * _N + src
    pad = _EPAD - _E
    idx = jnp.concatenate([idx, jnp.zeros((pad,), jnp.int32)])
    dstp = jnp.concatenate([dst, jnp.full((pad,), _N, jnp.int32)])
    idx_a, idx_b = idx[:_EHALF], idx[_EHALF:]
    dst_a, dst_b = dstp[:_EHALF], dstp[_EHALF:]

    wbig1 = jnp.concatenate(
        [W1.transpose(1, 0, 2).reshape(_D, _R * _D), loop1], axis=1)
    bbig1 = jnp.concatenate(
        [jnp.zeros((_R * _D,), jnp.float32), b1]).reshape(1, _WCOLS)
    wbig2 = jnp.concatenate(
        [W2.transpose(1, 0, 2).reshape(_D, _R * _D), loop2], axis=1)
    bbig2 = jnp.concatenate(
        [jnp.zeros((_R * _D,), jnp.float32), b2]).reshape(1, _WCOLS)

    tbl1, sl1 = _mm1(h, wbig1, bbig1)
    t1 = tbl1.reshape(_R * _N, _D)
    acc1a = _sc_segsum(t1, idx_a, dst_a)
    acc1b = _sc_segsum(t1, idx_b, dst_b)
    tbl2, sl2 = _mm2(acc1a, acc1b, sl1, wbig2, bbig2)
    t2 = tbl2.reshape(_R * _N, _D)
    acc2a = _sc_segsum(t2, idx_a, dst_a)
    acc2b = _sc_segsum(t2, idx_b, dst_b)
    part = _comb(acc2a, acc2b, sl2)
    return _head(part, Wc, bc.reshape(1, _C))


# speedup vs baseline: 2.3734x; 2.3734x over previous
import functools

import jax
import jax.numpy as jnp
from jax import lax
import jax.experimental.pallas as pl
from jax.experimental.pallas import tpu as pltpu
from jax.experimental.pallas import tpu_sc as plsc

_N = 10000
_E = 320000
_D = 128
_R = 8
_C = 16

_K = 128
_NI = 20
_EPW = _K * 4 * _NI
_EHALF = 16 * _EPW
_EPAD = 2 * _EHALF
_AROWS = 10240
_TROWS = _AROWS // 16

_RB = 1000
_GRID = _N // _RB


def _sc_body(tbl_ref, idx_ref, dst_ref, out_ref,
             ib0, ib1, ib2, ib3, db0, db1, db2, db3, rows0, rows1, acc,
             is0, is1, is2, is3, gs0, gs1, ss0, ss1):
    core = lax.axis_index("c")
    s = lax.axis_index("s")
    ebase = s * _EPW
    ni = _NI
    ibufs = [ib0, ib1, ib2, ib3]
    dbufs = [db0, db1, db2, db3]
    isems = [is0, is1, is2, is3]
    rows = [rows0, rows1]
    gsems = [gs0, gs1]
    ssems = [ss0, ss1]

    @pl.when(core == 0)
    def _only_core0():
        _sc_work(tbl_ref, idx_ref, dst_ref, out_ref, ibufs, dbufs, rows,
                 acc, isems, gsems, ssems, s, ebase, ni)


def _sc_work(tbl_ref, idx_ref, dst_ref, out_ref, ibufs, dbufs, rows, acc,
             isems, gsems, ssems, s, ebase, ni):
    rows0 = rows[0]
    def _zrow(r, carry):
        for l in range(8):
            rows0[r, pl.ds(l * 16, 16)] = jnp.zeros((16,), jnp.float32)
        return carry

    lax.fori_loop(0, _K, _zrow, 0)
    for kk in range(_TROWS // _K):
        pltpu.sync_copy(rows0, acc.at[pl.ds(s * _TROWS + kk * _K, _K)])
    plsc.subcore_barrier()

    def fire_idx(slot, ch):
        off = ebase + ch * _K
        pltpu.async_copy(idx_ref.at[pl.ds(off, _K)], ibufs[slot], isems[slot])
        pltpu.async_copy(dst_ref.at[pl.ds(off, _K)], dbufs[slot], isems[slot])

    def wait_idx(slot):
        pltpu.make_async_copy(
            idx_ref.at[pl.ds(0, _K)], ibufs[slot], isems[slot]).wait()
        pltpu.make_async_copy(
            dst_ref.at[pl.ds(0, _K)], dbufs[slot], isems[slot]).wait()

    def fire_g(slot, pr):
        pltpu.async_copy(tbl_ref.at[ibufs[slot]], rows[pr], gsems[pr])

    def wait_g(pr):
        pltpu.make_async_copy(tbl_ref.at[ibufs[0]], rows[pr],
                              gsems[pr]).wait()

    def fire_s(slot, pr):
        pltpu.async_copy(rows[pr], acc.at[dbufs[slot]], ssems[pr], add=True)

    def drain_s(pr):
        pltpu.make_async_copy(rows[pr], acc.at[dbufs[0]], ssems[pr]).wait()

    fire_idx(0, 0)
    fire_idx(1, 1)
    wait_idx(0)
    fire_g(0, 0)

    def body(i, carry):
        cb = 4 * i
        for j in range(4):
            pj = j % 2
            if j < 2:
                fire_idx((j + 2) % 4, cb + j + 2)
            else:
                @pl.when(i < ni - 1)
                def _(jj=j):
                    fire_idx((jj + 2) % 4, cb + jj + 2)
            if j == 0:
                @pl.when(i > 0)
                def _():
                    drain_s(1)
                wait_idx(1)
                fire_g(1, 1)
            elif j < 3:
                drain_s(1 - pj)
                wait_idx((j + 1) % 4)
                fire_g((j + 1) % 4, 1 - pj)
            else:
                @pl.when(i < ni - 1)
                def _():
                    drain_s(0)
                    wait_idx(0)
                    fire_g(0, 0)
            wait_g(pj)
            fire_s(j, pj)
        return carry

    lax.fori_loop(0, ni, body, 0)
    drain_s(0)
    drain_s(1)
    plsc.subcore_barrier()
    pltpu.sync_copy(acc.at[pl.ds(s * _TROWS, _TROWS)],
                    out_ref.at[pl.ds(s * _TROWS, _TROWS)])


_sc_segsum = pl.kernel(
    _sc_body,
    out_type=jax.ShapeDtypeStruct((_AROWS, _D), jnp.float32),
    mesh=plsc.VectorSubcoreMesh(core_axis_name="c", subcore_axis_name="s"),
    scratch_types=(
        [pltpu.VMEM((_K,), jnp.int32)] * 8
        + [pltpu.VMEM((_K, _D), jnp.float32)] * 2
        + [pltpu.VMEM_SHARED((_AROWS, _D), jnp.float32)]
        + [pltpu.SemaphoreType.DMA] * 8
    ),
)


def _mm1_body(x_ref, w_ref, b_ref, tbl_ref, sl_ref):
    res = jnp.dot(x_ref[...], w_ref[...],
                  preferred_element_type=jnp.float32) + b_ref[...]
    for r in range(_R):
        tbl_ref[r] = res[:, r * _D:(r + 1) * _D]
    sl_ref[...] = res[:, _R * _D:]


def _mm2_body(acca_ref, accb_ref, sl_ref, w_ref, b_ref, tbl_ref, sl2_ref):
    x = jnp.maximum(acca_ref[...] + accb_ref[...] + sl_ref[...], 0.0)
    res = jnp.dot(x, w_ref[...],
                  preferred_element_type=jnp.float32) + b_ref[...]
    for r in range(_R):
        tbl_ref[r] = res[:, r * _D:(r + 1) * _D]
    sl2_ref[...] = res[:, _R * _D:]


def _comb_body(acca_ref, accb_ref, sl_ref, out_ref):
    y = jnp.maximum(acca_ref[...] + accb_ref[...] + sl_ref[...], 0.0)
    out_ref[...] = jnp.sum(y, axis=0, keepdims=True)[None]


def _head_body(p_ref, wc_ref, bc_ref, out_ref):
    hg = jnp.sum(p_ref[..., 0, :], axis=0, keepdims=True) * (1.0 / _N)
    logits = jnp.dot(hg, wc_ref[...],
                     preferred_element_type=jnp.float32) + bc_ref[...]
    out_ref[...] = jax.nn.softmax(logits, axis=1)


_WCOLS = _R * _D + _D

_mm1 = pl.pallas_call(
    _mm1_body,
    grid=(_GRID,),
    in_specs=[
        pl.BlockSpec((_RB, _D), lambda i: (i, 0)),
        pl.BlockSpec((_D, _WCOLS), lambda i: (0, 0)),
        pl.BlockSpec((1, _WCOLS), lambda i: (0, 0)),
    ],
    out_specs=[
        pl.BlockSpec((_R, _RB, _D), lambda i: (0, i, 0)),
        pl.BlockSpec((_RB, _D), lambda i: (i, 0)),
    ],
    out_shape=[
        jax.ShapeDtypeStruct((_R, _N, _D), jnp.float32),
        jax.ShapeDtypeStruct((_N, _D), jnp.float32),
    ],
)

_mm2 = pl.pallas_call(
    _mm2_body,
    grid=(_GRID,),
    in_specs=[
        pl.BlockSpec((_RB, _D), lambda i: (i, 0)),
        pl.BlockSpec((_RB, _D), lambda i: (i, 0)),
        pl.BlockSpec((_RB, _D), lambda i: (i, 0)),
        pl.BlockSpec((_D, _WCOLS), lambda i: (0, 0)),
        pl.BlockSpec((1, _WCOLS), lambda i: (0, 0)),
    ],
    out_specs=[
        pl.BlockSpec((_R, _RB, _D), lambda i: (0, i, 0)),
        pl.BlockSpec((_RB, _D), lambda i: (i, 0)),
    ],
    out_shape=[
        jax.ShapeDtypeStruct((_R, _N, _D), jnp.float32),
        jax.ShapeDtypeStruct((_N, _D), jnp.float32),
    ],
)

_comb = pl.pallas_call(
    _comb_body,
    grid=(_GRID,),
    in_specs=[
        pl.BlockSpec((_RB, _D), lambda i: (i, 0)),
        pl.BlockSpec((_RB, _D), lambda i: (i, 0)),
        pl.BlockSpec((_RB, _D), lambda i: (i, 0)),
    ],
    out_specs=pl.BlockSpec((1, 1, _D), lambda i: (i, 0, 0)),
    out_shape=jax.ShapeDtypeStruct((_GRID, 1, _D), jnp.float32),
)

_head = pl.pallas_call(
    _head_body,
    out_shape=jax.ShapeDtypeStruct((1, _C), jnp.float32),
)


def kernel(h, edge_index, rel_types, W1, loop1, b1, W2, loop2, b2, Wc, bc):
    src = edge_index[0].astype(jnp.int32)
    dst = edge_index[1].astype(jnp.int32)
    rel = rel_types.astype(jnp.int32)
    idx = rel * _N + src
    pad = _EPAD - _E
    pidx = jnp.arange(pad, dtype=jnp.int32)
    idx = jnp.concatenate([idx, pidx % (_R * _N)])
    dstp = jnp.concatenate([dst, _N + (pidx % (_AROWS - _N))])
    idx_a, idx_b = idx[:_EHALF], idx[_EHALF:]
    dst_a, dst_b = dstp[:_EHALF], dstp[_EHALF:]

    wbig1 = jnp.concatenate(
        [W1.transpose(1, 0, 2).reshape(_D, _R * _D), loop1], axis=1)
    bbig1 = jnp.concatenate(
        [jnp.zeros((_R * _D,), jnp.float32), b1]).reshape(1, _WCOLS)
    wbig2 = jnp.concatenate(
        [W2.transpose(1, 0, 2).reshape(_D, _R * _D), loop2], axis=1)
    bbig2 = jnp.concatenate(
        [jnp.zeros((_R * _D,), jnp.float32), b2]).reshape(1, _WCOLS)

    tbl1, sl1 = _mm1(h, wbig1, bbig1)
    t1 = tbl1.reshape(_R * _N, _D)
    acc1a = _sc_segsum(t1, idx_a, dst_a)
    acc1b = _sc_segsum(t1, idx_b, dst_b)
    tbl2, sl2 = _mm2(acc1a, acc1b, sl1, wbig2, bbig2)
    t2 = tbl2.reshape(_R * _N, _D)
    acc2a = _sc_segsum(t2, idx_a, dst_a)
    acc2b = _sc_segsum(t2, idx_b, dst_b)
    part = _comb(acc2a, acc2b, sl2)
    return _head(part, Wc, bc.reshape(1, _C))


# speedup vs baseline: 2.5533x; 1.0758x over previous
import functools

import jax
import jax.numpy as jnp
from jax import lax
import jax.experimental.pallas as pl
from jax.experimental.pallas import tpu as pltpu
from jax.experimental.pallas import tpu_sc as plsc

_N = 10000
_E = 320000
_D = 128
_R = 8
_C = 16

_K = 128
_NI = 40
_EPW = _K * 4 * _NI
_EPAD = 16 * _EPW
_AROWS = 10240
_TROWS = _AROWS // 16

_RB = 1000
_GRID = _N // _RB


def _sc_body(tbl_ref, idx_ref, dst_ref, out_ref,
             ib0, ib1, ib2, ib3, db0, db1, db2, db3, rows0, rows1, acc,
             is0, is1, is2, is3, gs0, gs1, ss0, ss1):
    core = lax.axis_index("c")
    s = lax.axis_index("s")
    ebase = s * _EPW
    ni = _NI
    ibufs = [ib0, ib1, ib2, ib3]
    dbufs = [db0, db1, db2, db3]
    isems = [is0, is1, is2, is3]
    rows = [rows0, rows1]
    gsems = [gs0, gs1]
    ssems = [ss0, ss1]

    @pl.when(core == 0)
    def _only_core0():
        _sc_work(tbl_ref, idx_ref, dst_ref, out_ref, ibufs, dbufs, rows,
                 acc, isems, gsems, ssems, s, ebase, ni)


def _sc_work(tbl_ref, idx_ref, dst_ref, out_ref, ibufs, dbufs, rows, acc,
             isems, gsems, ssems, s, ebase, ni):
    rows0 = rows[0]
    def _zrow(r, carry):
        for l in range(8):
            rows0[r, pl.ds(l * 16, 16)] = jnp.zeros((16,), jnp.float32)
        return carry

    lax.fori_loop(0, _K, _zrow, 0)
    for kk in range(_TROWS // _K):
        pltpu.sync_copy(rows0, acc.at[pl.ds(s * _TROWS + kk * _K, _K)])
    plsc.subcore_barrier()

    def fire_idx(slot, ch):
        off = ebase + ch * _K
        pltpu.async_copy(idx_ref.at[pl.ds(off, _K)], ibufs[slot], isems[slot])
        pltpu.async_copy(dst_ref.at[pl.ds(off, _K)], dbufs[slot], isems[slot])

    def wait_idx(slot):
        pltpu.make_async_copy(
            idx_ref.at[pl.ds(0, _K)], ibufs[slot], isems[slot]).wait()
        pltpu.make_async_copy(
            dst_ref.at[pl.ds(0, _K)], dbufs[slot], isems[slot]).wait()

    def fire_g(slot, pr):
        pltpu.async_copy(tbl_ref.at[ibufs[slot]], rows[pr], gsems[pr])

    def wait_g(pr):
        pltpu.make_async_copy(tbl_ref.at[ibufs[0]], rows[pr],
                              gsems[pr]).wait()

    def fire_s(slot, pr):
        pltpu.async_copy(rows[pr], acc.at[dbufs[slot]], ssems[pr], add=True)

    def drain_s(pr):
        pltpu.make_async_copy(rows[pr], acc.at[dbufs[0]], ssems[pr]).wait()

    fire_idx(0, 0)
    fire_idx(1, 1)
    wait_idx(0)
    fire_g(0, 0)

    def body(i, carry):
        cb = 4 * i
        for j in range(4):
            pj = j % 2
            if j < 2:
                fire_idx((j + 2) % 4, cb + j + 2)
            else:
                @pl.when(i < ni - 1)
                def _(jj=j):
                    fire_idx((jj + 2) % 4, cb + jj + 2)
            if j == 0:
                @pl.when(i > 0)
                def _():
                    drain_s(1)
                wait_idx(1)
                fire_g(1, 1)
            elif j < 3:
                drain_s(1 - pj)
                wait_idx((j + 1) % 4)
                fire_g((j + 1) % 4, 1 - pj)
            else:
                @pl.when(i < ni - 1)
                def _():
                    drain_s(0)
                    wait_idx(0)
                    fire_g(0, 0)
            wait_g(pj)
            fire_s(j, pj)
        return carry

    lax.fori_loop(0, ni, body, 0)
    drain_s(0)
    drain_s(1)
    plsc.subcore_barrier()
    pltpu.sync_copy(acc.at[pl.ds(s * _TROWS, _TROWS)],
                    out_ref.at[pl.ds(s * _TROWS, _TROWS)])


_sc_segsum = pl.kernel(
    _sc_body,
    out_type=jax.ShapeDtypeStruct((_AROWS, _D), jnp.float32),
    mesh=plsc.VectorSubcoreMesh(core_axis_name="c", subcore_axis_name="s"),
    scratch_types=(
        [pltpu.VMEM((_K,), jnp.int32)] * 8
        + [pltpu.VMEM((_K, _D), jnp.float32)] * 2
        + [pltpu.VMEM_SHARED((_AROWS, _D), jnp.float32)]
        + [pltpu.SemaphoreType.DMA] * 8
    ),
)


def _mm1_body(x_ref, w_ref, b_ref, tbl_ref, sl_ref):
    res = jnp.dot(x_ref[...], w_ref[...],
                  preferred_element_type=jnp.float32) + b_ref[...]
    for r in range(_R):
        tbl_ref[r] = res[:, r * _D:(r + 1) * _D]
    sl_ref[...] = res[:, _R * _D:]


def _mm2_body(acc_ref, sl_ref, w_ref, b_ref, tbl_ref, sl2_ref):
    x = jnp.maximum(acc_ref[...] + sl_ref[...], 0.0)
    res = jnp.dot(x, w_ref[...],
                  preferred_element_type=jnp.float32) + b_ref[...]
    for r in range(_R):
        tbl_ref[r] = res[:, r * _D:(r + 1) * _D]
    sl2_ref[...] = res[:, _R * _D:]


def _comb_body(acc_ref, sl_ref, out_ref):
    y = jnp.maximum(acc_ref[...] + sl_ref[...], 0.0)
    out_ref[...] = jnp.sum(y, axis=0, keepdims=True)[None]


def _head_body(p_ref, wc_ref, bc_ref, out_ref):
    hg = jnp.sum(p_ref[..., 0, :], axis=0, keepdims=True) * (1.0 / _N)
    logits = jnp.dot(hg, wc_ref[...],
                     preferred_element_type=jnp.float32) + bc_ref[...]
    out_ref[...] = jax.nn.softmax(logits, axis=1)


_WCOLS = _R * _D + _D

_mm1 = pl.pallas_call(
    _mm1_body,
    grid=(_GRID,),
    in_specs=[
        pl.BlockSpec((_RB, _D), lambda i: (i, 0)),
        pl.BlockSpec((_D, _WCOLS), lambda i: (0, 0)),
        pl.BlockSpec((1, _WCOLS), lambda i: (0, 0)),
    ],
    out_specs=[
        pl.BlockSpec((_R, _RB, _D), lambda i: (0, i, 0)),
        pl.BlockSpec((_RB, _D), lambda i: (i, 0)),
    ],
    out_shape=[
        jax.ShapeDtypeStruct((_R, _N, _D), jnp.float32),
        jax.ShapeDtypeStruct((_N, _D), jnp.float32),
    ],
)

_mm2 = pl.pallas_call(
    _mm2_body,
    grid=(_GRID,),
    in_specs=[
        pl.BlockSpec((_RB, _D), lambda i: (i, 0)),
        pl.BlockSpec((_RB, _D), lambda i: (i, 0)),
        pl.BlockSpec((_D, _WCOLS), lambda i: (0, 0)),
        pl.BlockSpec((1, _WCOLS), lambda i: (0, 0)),
    ],
    out_specs=[
        pl.BlockSpec((_R, _RB, _D), lambda i: (0, i, 0)),
        pl.BlockSpec((_RB, _D), lambda i: (i, 0)),
    ],
    out_shape=[
        jax.ShapeDtypeStruct((_R, _N, _D), jnp.float32),
        jax.ShapeDtypeStruct((_N, _D), jnp.float32),
    ],
)

_comb = pl.pallas_call(
    _comb_body,
    grid=(_GRID,),
    in_specs=[
        pl.BlockSpec((_RB, _D), lambda i: (i, 0)),
        pl.BlockSpec((_RB, _D), lambda i: (i, 0)),
    ],
    out_specs=pl.BlockSpec((1, 1, _D), lambda i: (i, 0, 0)),
    out_shape=jax.ShapeDtypeStruct((_GRID, 1, _D), jnp.float32),
)

_head = pl.pallas_call(
    _head_body,
    out_shape=jax.ShapeDtypeStruct((1, _C), jnp.float32),
)


def kernel(h, edge_index, rel_types, W1, loop1, b1, W2, loop2, b2, Wc, bc):
    src = edge_index[0].astype(jnp.int32)
    dst = edge_index[1].astype(jnp.int32)
    rel = rel_types.astype(jnp.int32)
    idx = rel * _N + src
    pad = _EPAD - _E
    pidx = jnp.arange(pad, dtype=jnp.int32)
    idx = jnp.concatenate([idx, pidx % (_R * _N)])
    dstp = jnp.concatenate([dst, _N + (pidx % (_AROWS - _N))])

    wbig1 = jnp.concatenate(
        [W1.transpose(1, 0, 2).reshape(_D, _R * _D), loop1], axis=1)
    bbig1 = jnp.concatenate(
        [jnp.zeros((_R * _D,), jnp.float32), b1]).reshape(1, _WCOLS)
    wbig2 = jnp.concatenate(
        [W2.transpose(1, 0, 2).reshape(_D, _R * _D), loop2], axis=1)
    bbig2 = jnp.concatenate(
        [jnp.zeros((_R * _D,), jnp.float32), b2]).reshape(1, _WCOLS)

    tbl1, sl1 = _mm1(h, wbig1, bbig1)
    acc1 = _sc_segsum(tbl1.reshape(_R * _N, _D), idx, dstp)
    tbl2, sl2 = _mm2(acc1, sl1, wbig2, bbig2)
    acc2 = _sc_segsum(tbl2.reshape(_R * _N, _D), idx, dstp)
    part = _comb(acc2, sl2)
    return _head(part, Wc, bc.reshape(1, _C))
